# baseline (speedup 1.0000x reference)
import functools

import jax
import jax.numpy as jnp
from jax import lax
from jax.experimental import pallas as pl
from jax.experimental.pallas import tpu as pltpu
from jax.experimental.pallas import tpu_sc as plsc

N = 10000
E = 320000
D = 128
H = 128
C = 10
ALPHA = 0.1

NC = 2
NS = 16
NW = NC * NS
LANE = 16

EPW = E // NW
TCH = E // 128
CPW = 80
SST = 16
NSTG = CPW // SST
PADROWS = CPW * NW
NPAD = 10240
RPT = NPAD // NS

B = 1024
G = -(-N // B)

_mesh = plsc.VectorSubcoreMesh(
    core_axis_name="c", subcore_axis_name="s", num_cores=NC, num_subcores=NS)


@functools.partial(
    pl.kernel,
    out_type=(jax.ShapeDtypeStruct((NW * N,), jnp.float32),
              jax.ShapeDtypeStruct((NW * N,), jnp.float32)),
    mesh=_mesh,
    compiler_params=pltpu.CompilerParams(needs_layout_passes=False),
    scratch_types=[
        pltpu.VMEM((EPW,), jnp.int32),
        pltpu.VMEM((EPW,), jnp.int32),
        pltpu.VMEM((N,), jnp.float32),
        pltpu.VMEM((N,), jnp.float32),
    ],
)
def _deg_kernel(src_hbm, dst_hbm, do_hbm, di_hbm, sidx, didx, dout, din):
    cid = lax.axis_index("c")
    sid = lax.axis_index("s")
    wid = cid * NS + sid
    base = wid * EPW
    pltpu.sync_copy(src_hbm.at[pl.ds(base, EPW)], sidx)
    pltpu.sync_copy(dst_hbm.at[pl.ds(base, EPW)], didx)

    zeros = jnp.zeros((LANE,), jnp.float32)

    def zero_body(i, carry):
        dout[pl.ds(i * LANE, LANE)] = zeros
        din[pl.ds(i * LANE, LANE)] = zeros
        return carry

    lax.fori_loop(0, N // LANE, zero_body, 0)

    ones = jnp.ones((LANE,), jnp.float32)

    def body(k, carry):
        si = sidx[pl.ds(k * LANE, LANE)]
        plsc.addupdate_scatter(dout, [si], ones)
        di = didx[pl.ds(k * LANE, LANE)]
        plsc.addupdate_scatter(din, [di], ones)
        return carry

    lax.fori_loop(0, EPW // LANE, body, 0)

    pltpu.sync_copy(dout, do_hbm.at[pl.ds(wid * N, N)])
    pltpu.sync_copy(din, di_hbm.at[pl.ds(wid * N, N)])


@functools.partial(
    pl.kernel,
    out_type=jax.ShapeDtypeStruct((NC, NPAD, H), jnp.float32),
    mesh=_mesh,
    compiler_params=pltpu.CompilerParams(needs_layout_passes=False),
    scratch_types=[
        pltpu.VMEM_SHARED((NPAD, H), jnp.float32),
        pltpu.VMEM((SST, 128), jnp.int32),
        pltpu.VMEM((SST, 128), jnp.int32),
        pltpu.VMEM((2, 128, H), jnp.float32),
        pltpu.SemaphoreType.DMA((2,)),
        pltpu.SemaphoreType.DMA((2,)),
    ],
)
def _agg_kernel(y_hbm, src_hbm, dst_hbm, zer_hbm, out_hbm,
                acc, srows, drows, rbuf, sem, ssem):
    cid = lax.axis_index("c")
    sid = lax.axis_index("s")
    wid = cid * NS + sid
    n_w = jnp.clip(TCH - wid * CPW, 0, CPW)

    pltpu.sync_copy(zer_hbm, acc.at[pl.ds(sid * RPT, RPT)])
    plsc.subcore_barrier()

    def stage(t, carry):
        row0 = wid * CPW + t * SST
        pltpu.sync_copy(src_hbm.at[pl.ds(row0, SST)], srows)
        pltpu.sync_copy(dst_hbm.at[pl.ds(row0, SST)], drows)
        n_t = jnp.clip(n_w - t * SST, 0, SST)

        @pl.when(n_t > 0)
        def _():
            pltpu.async_copy(y_hbm.at[srows.at[0]], rbuf.at[0], sem.at[0])

        def body(j, c2):
            p = lax.rem(j, 2)
            q = lax.rem(j + 1, 2)

            pltpu.make_async_copy(y_hbm.at[srows.at[j]], rbuf.at[p],
                                  sem.at[p]).wait()
            pltpu.async_copy(rbuf.at[p], acc.at[drows.at[j]], ssem.at[p],
                             add=True)

            @pl.when(j + 1 < n_t)
            def _():
                @pl.when(j >= 1)
                def _():
                    pltpu.make_async_copy(rbuf.at[q], acc.at[drows.at[j - 1]],
                                          ssem.at[q]).wait()

                pltpu.async_copy(y_hbm.at[srows.at[j + 1]], rbuf.at[q],
                                 sem.at[q])
            return c2

        lax.fori_loop(0, n_t, body, 0)

        @pl.when(n_t > 1)
        def _():
            pltpu.make_async_copy(rbuf.at[lax.rem(n_t - 2, 2)],
                                  acc.at[drows.at[0]],
                                  ssem.at[lax.rem(n_t - 2, 2)]).wait()

        @pl.when(n_t > 0)
        def _():
            pltpu.make_async_copy(rbuf.at[lax.rem(n_t - 1, 2)],
                                  acc.at[drows.at[0]],
                                  ssem.at[lax.rem(n_t - 1, 2)]).wait()

        return carry

    lax.fori_loop(0, NSTG, stage, 0)
    plsc.subcore_barrier()
    pltpu.sync_copy(acc.at[pl.ds(sid * RPT, RPT)],
                    out_hbm.at[cid, pl.ds(sid * RPT, RPT)])


@functools.partial(
    pl.kernel,
    out_type=(jax.ShapeDtypeStruct((E,), jnp.float32),
              jax.ShapeDtypeStruct((NW * N,), jnp.float32)),
    mesh=_mesh,
    compiler_params=pltpu.CompilerParams(needs_layout_passes=False),
    scratch_types=[
        pltpu.VMEM((EPW,), jnp.int32),
        pltpu.VMEM((EPW,), jnp.int32),
        pltpu.VMEM((EPW,), jnp.float32),
        pltpu.VMEM((N,), jnp.float32),
        pltpu.VMEM((N,), jnp.float32),
        pltpu.VMEM((N,), jnp.float32),
    ],
)
def _z_kernel(es_hbm, ed_hbm, src_hbm, dst_hbm, z_hbm, den_hbm,
              sidx, didx, zbuf, esv, edv, denv):
    cid = lax.axis_index("c")
    sid = lax.axis_index("s")
    wid = cid * NS + sid
    base = wid * EPW
    pltpu.sync_copy(src_hbm.at[pl.ds(base, EPW)], sidx)
    pltpu.sync_copy(dst_hbm.at[pl.ds(base, EPW)], didx)
    pltpu.sync_copy(es_hbm, esv)
    pltpu.sync_copy(ed_hbm, edv)

    zeros = jnp.zeros((LANE,), jnp.float32)

    def zden(i, carry):
        denv[pl.ds(i * LANE, LANE)] = zeros
        return carry

    lax.fori_loop(0, N // LANE, zden, 0)

    neg = jnp.full((LANE,), -3.0e38, jnp.float32)

    def mx(i, carry):
        a, b = carry
        return (jnp.maximum(a, esv[pl.ds(i * LANE, LANE)]),
                jnp.maximum(b, edv[pl.ds(i * LANE, LANE)]))

    am, bm = lax.fori_loop(0, N // LANE, mx, (neg, neg))
    s_max = jnp.max(am) + jnp.max(bm)
    g_bound = jnp.maximum(s_max, ALPHA * s_max)

    def body(k, carry):
        sl = pl.ds(k * LANE, LANE)
        si = sidx[sl]
        di = didx[sl]
        s = plsc.load_gather(esv, [si]) + plsc.load_gather(edv, [di])
        e = jnp.maximum(s, ALPHA * s)
        z = jnp.exp(e - g_bound)
        zbuf[sl] = z
        plsc.addupdate_scatter(denv, [di], z)
        return carry

    lax.fori_loop(0, EPW // LANE, body, 0)

    pltpu.sync_copy(zbuf, z_hbm.at[pl.ds(base, EPW)])
    pltpu.sync_copy(denv, den_hbm.at[pl.ds(wid * N, N)])


@functools.partial(
    pl.kernel,
    out_type=jax.ShapeDtypeStruct((NC, NPAD, H), jnp.float32),
    mesh=_mesh,
    compiler_params=pltpu.CompilerParams(needs_layout_passes=False),
    scratch_types=[
        pltpu.VMEM_SHARED((NPAD, H), jnp.float32),
        pltpu.VMEM((SST, 128), jnp.int32),
        pltpu.VMEM((SST, 128), jnp.int32),
        pltpu.VMEM((SST, 128), jnp.float32),
        pltpu.VMEM((2, 128, H), jnp.float32),
        pltpu.SemaphoreType.DMA((2,)),
        pltpu.SemaphoreType.DMA((2,)),
    ],
)
def _wagg_kernel(wh_hbm, src_hbm, dst_hbm, z_hbm, zer_hbm, out_hbm,
                 acc, srows, drows, zrows, rbuf, sem, ssem):
    cid = lax.axis_index("c")
    sid = lax.axis_index("s")
    wid = cid * NS + sid
    n_w = jnp.clip(TCH - wid * CPW, 0, CPW)

    pltpu.sync_copy(zer_hbm, acc.at[pl.ds(sid * RPT, RPT)])
    plsc.subcore_barrier()

    def stage(t, carry):
        row0 = wid * CPW + t * SST
        pltpu.sync_copy(src_hbm.at[pl.ds(row0, SST)], srows)
        pltpu.sync_copy(dst_hbm.at[pl.ds(row0, SST)], drows)
        pltpu.sync_copy(z_hbm.at[pl.ds(row0, SST)], zrows)
        n_t = jnp.clip(n_w - t * SST, 0, SST)

        @pl.when(n_t > 0)
        def _():
            pltpu.async_copy(wh_hbm.at[srows.at[0]], rbuf.at[0], sem.at[0])

        def body(j, c2):
            p = lax.rem(j, 2)
            q = lax.rem(j + 1, 2)

            pltpu.make_async_copy(wh_hbm.at[srows.at[j]], rbuf.at[p],
                                  sem.at[p]).wait()
            jv = jnp.full((LANE,), j, jnp.int32)

            def scale(r4, c3):
                for rr in range(4):
                    r = r4 * 4 + rr
                    zr = plsc.load_gather(
                        zrows, [jv, jnp.full((LANE,), r, jnp.int32)])
                    for cc in range(H // LANE):
                        cs = pl.ds(cc * LANE, LANE)
                        rbuf[p, r, cs] = rbuf[p, r, cs] * zr
                return c3

            lax.fori_loop(0, 32, scale, 0)
            pltpu.async_copy(rbuf.at[p], acc.at[drows.at[j]], ssem.at[p],
                             add=True)

            @pl.when(j + 1 < n_t)
            def _():
                @pl.when(j >= 1)
                def _():
                    pltpu.make_async_copy(rbuf.at[q], acc.at[drows.at[j - 1]],
                                          ssem.at[q]).wait()

                pltpu.async_copy(wh_hbm.at[srows.at[j + 1]], rbuf.at[q],
                                 sem.at[q])
            return c2

        lax.fori_loop(0, n_t, body, 0)

        @pl.when(n_t > 1)
        def _():
            pltpu.make_async_copy(rbuf.at[lax.rem(n_t - 2, 2)],
                                  acc.at[drows.at[0]],
                                  ssem.at[lax.rem(n_t - 2, 2)]).wait()

        @pl.when(n_t > 0)
        def _():
            pltpu.make_async_copy(rbuf.at[lax.rem(n_t - 1, 2)],
                                  acc.at[drows.at[0]],
                                  ssem.at[lax.rem(n_t - 1, 2)]).wait()

        return carry

    lax.fori_loop(0, NSTG, stage, 0)
    plsc.subcore_barrier()
    pltpu.sync_copy(acc.at[pl.ds(sid * RPT, RPT)],
                    out_hbm.at[cid, pl.ds(sid * RPT, RPT)])


def _gcn1_tc(x_ref, degp_ref, w1_ref, y_ref):
    deg = jnp.sum(degp_ref[...], axis=0)
    scale = lax.rsqrt(jnp.maximum(deg, 1.0))
    xs = x_ref[...] * scale[:, None]
    y_ref[...] = jnp.dot(xs, w1_ref[...], preferred_element_type=jnp.float32)


def _gat_tc(aggp_ref, degp_ref, w2_ref, as_ref, ad_ref,
            wh_ref, es_ref, ed_ref):
    agg = aggp_ref[0] + aggp_ref[1]
    deg = jnp.sum(degp_ref[...], axis=0)
    h = jnp.maximum(agg * lax.rsqrt(jnp.maximum(deg, 1.0))[:, None], 0.0)
    wh = jnp.dot(h, w2_ref[...], preferred_element_type=jnp.float32)
    wh_ref[...] = wh
    es_ref[...] = jnp.broadcast_to(jnp.dot(wh, as_ref[...])[None, None, :],
                                   (1, 8, B))
    ed_ref[...] = jnp.broadcast_to(jnp.dot(wh, ad_ref[...])[None, None, :],
                                   (1, 8, B))


def _head_tc(h2p_ref, denp_ref, wp_ref, bp_ref, out_ref):
    h2 = h2p_ref[0] + h2p_ref[1]
    den = jnp.sum(denp_ref[...], axis=0)
    h2 = h2 * (1.0 / (den + 1e-16))[:, None]
    h2 = jnp.where(h2 > 0.0, h2, jnp.exp(h2) - 1.0)
    out_ref[...] = (jnp.dot(h2, wp_ref[...], preferred_element_type=jnp.float32)
                    + bp_ref[...][None, :])


def kernel(x, edge_index, W1, W2, a_src, a_dst, Wp, bp):
    src = edge_index[0]
    dst = edge_index[1]
    src2d = jnp.pad(src.reshape(TCH, 128), ((0, PADROWS - TCH), (0, 0)))
    dst2d = jnp.pad(dst.reshape(TCH, 128), ((0, PADROWS - TCH), (0, 0)))
    zer = jnp.zeros((RPT, H), jnp.float32)

    dof, dif = _deg_kernel(src, dst)
    dop = dof.reshape(NW, N)
    dip = dif.reshape(NW, N)

    y = pl.pallas_call(
        _gcn1_tc,
        grid=(G,),
        in_specs=[
            pl.BlockSpec((B, D), lambda g: (g, 0)),
            pl.BlockSpec((NW, B), lambda g: (0, g)),
            pl.BlockSpec((D, H), lambda g: (0, 0)),
        ],
        out_specs=pl.BlockSpec((B, H), lambda g: (g, 0)),
        out_shape=jax.ShapeDtypeStruct((N, H), jnp.float32),
    )(x, dop, W1)

    aggp = _agg_kernel(y, src2d, dst2d, zer)

    wh, es2, ed2 = pl.pallas_call(
        _gat_tc,
        grid=(G,),
        in_specs=[
            pl.BlockSpec((NC, B, H), lambda g: (0, g, 0)),
            pl.BlockSpec((NW, B), lambda g: (0, g)),
            pl.BlockSpec((H, H), lambda g: (0, 0)),
            pl.BlockSpec((H,), lambda g: (0,)),
            pl.BlockSpec((H,), lambda g: (0,)),
        ],
        out_specs=[
            pl.BlockSpec((B, H), lambda g: (g, 0)),
            pl.BlockSpec((1, 8, B), lambda g: (g, 0, 0)),
            pl.BlockSpec((1, 8, B), lambda g: (g, 0, 0)),
        ],
        out_shape=[
            jax.ShapeDtypeStruct((N, H), jnp.float32),
            jax.ShapeDtypeStruct((G, 8, B), jnp.float32),
            jax.ShapeDtypeStruct((G, 8, B), jnp.float32),
        ],
    )(aggp, dip, W2, a_src, a_dst)

    es = es2[:, 0, :].reshape(G * B)[:N]
    ed = ed2[:, 0, :].reshape(G * B)[:N]

    zf, denf = _z_kernel(es, ed, src, dst)
    denp = denf.reshape(NW, N)
    z2d = jnp.pad(zf.reshape(TCH, 128), ((0, PADROWS - TCH), (0, 0)))

    h2p = _wagg_kernel(wh, src2d, dst2d, z2d, zer)

    logits = pl.pallas_call(
        _head_tc,
        grid=(G,),
        in_specs=[
            pl.BlockSpec((NC, B, H), lambda g: (0, g, 0)),
            pl.BlockSpec((NW, B), lambda g: (0, g)),
            pl.BlockSpec((H, C), lambda g: (0, 0)),
            pl.BlockSpec((C,), lambda g: (0,)),
        ],
        out_specs=pl.BlockSpec((B, C), lambda g: (g, 0)),
        out_shape=jax.ShapeDtypeStruct((N, C), jnp.float32),
    )(h2p, denp, Wp, bp)

    return logits


# speedup vs baseline: 1.1593x; 1.1593x over previous
import functools

import jax
import jax.numpy as jnp
from jax import lax
from jax.experimental import pallas as pl
from jax.experimental.pallas import tpu as pltpu
from jax.experimental.pallas import tpu_sc as plsc

N = 10000
E = 320000
D = 128
H = 128
C = 10
ALPHA = 0.1

NC = 2
NS = 16
NW = NC * NS
LANE = 16

EPW = E // NW
TCH = E // 128
CPW = 80
SST = 16
NSTG = CPW // SST
PADROWS = CPW * NW
NPAD = 10240
RPT = NPAD // NS

B = 1024
G = -(-N // B)

_mesh = plsc.VectorSubcoreMesh(
    core_axis_name="c", subcore_axis_name="s", num_cores=NC, num_subcores=NS)


@functools.partial(
    pl.kernel,
    out_type=(jax.ShapeDtypeStruct((NW * N,), jnp.float32),
              jax.ShapeDtypeStruct((NW * N,), jnp.float32)),
    mesh=_mesh,
    compiler_params=pltpu.CompilerParams(needs_layout_passes=False),
    scratch_types=[
        pltpu.VMEM((EPW,), jnp.int32),
        pltpu.VMEM((EPW,), jnp.int32),
        pltpu.VMEM((N,), jnp.float32),
        pltpu.VMEM((N,), jnp.float32),
    ],
)
def _deg_kernel(src_hbm, dst_hbm, do_hbm, di_hbm, sidx, didx, dout, din):
    cid = lax.axis_index("c")
    sid = lax.axis_index("s")
    wid = cid * NS + sid
    base = wid * EPW
    pltpu.sync_copy(src_hbm.at[pl.ds(base, EPW)], sidx)
    pltpu.sync_copy(dst_hbm.at[pl.ds(base, EPW)], didx)

    zeros = jnp.zeros((LANE,), jnp.float32)

    def zero_body(i, carry):
        dout[pl.ds(i * LANE, LANE)] = zeros
        din[pl.ds(i * LANE, LANE)] = zeros
        return carry

    lax.fori_loop(0, N // LANE, zero_body, 0)

    ones = jnp.ones((LANE,), jnp.float32)

    def body(k, carry):
        si = sidx[pl.ds(k * LANE, LANE)]
        plsc.addupdate_scatter(dout, [si], ones)
        di = didx[pl.ds(k * LANE, LANE)]
        plsc.addupdate_scatter(din, [di], ones)
        return carry

    lax.fori_loop(0, EPW // LANE, body, 0)

    pltpu.sync_copy(dout, do_hbm.at[pl.ds(wid * N, N)])
    pltpu.sync_copy(din, di_hbm.at[pl.ds(wid * N, N)])


@functools.partial(
    pl.kernel,
    out_type=jax.ShapeDtypeStruct((NC, NPAD, H), jnp.float32),
    mesh=_mesh,
    compiler_params=pltpu.CompilerParams(needs_layout_passes=False),
    scratch_types=[
        pltpu.VMEM_SHARED((NPAD, H), jnp.float32),
        pltpu.VMEM((SST, 128), jnp.int32),
        pltpu.VMEM((SST, 128), jnp.int32),
        pltpu.VMEM((2, 128, H), jnp.float32),
        pltpu.SemaphoreType.DMA((2,)),
    ],
)
def _agg_kernel(y_hbm, src_hbm, dst_hbm, zer_hbm, out_hbm,
                acc, srows, drows, rbuf, sem):
    cid = lax.axis_index("c")
    sid = lax.axis_index("s")
    wid = cid * NS + sid
    n_w = jnp.clip(TCH - wid * CPW, 0, CPW)

    pltpu.sync_copy(zer_hbm, acc.at[pl.ds(sid * RPT, RPT)])
    plsc.subcore_barrier()

    def stage(t, carry):
        row0 = wid * CPW + t * SST
        pltpu.sync_copy(src_hbm.at[pl.ds(row0, SST)], srows)
        pltpu.sync_copy(dst_hbm.at[pl.ds(row0, SST)], drows)
        n_t = jnp.clip(n_w - t * SST, 0, SST)

        @pl.when(n_t > 0)
        def _():
            pltpu.async_copy(y_hbm.at[srows.at[0]], rbuf.at[0], sem.at[0])

        def body(j, c2):
            p = lax.rem(j, 2)
            q = lax.rem(j + 1, 2)

            @pl.when(j + 1 < n_t)
            def _():
                pltpu.async_copy(y_hbm.at[srows.at[j + 1]], rbuf.at[q],
                                 sem.at[q])

            pltpu.make_async_copy(y_hbm.at[srows.at[j]], rbuf.at[p],
                                  sem.at[p]).wait()
            pltpu.sync_copy(rbuf.at[p], acc.at[drows.at[j]], add=True)
            return c2

        lax.fori_loop(0, n_t, body, 0)
        return carry

    lax.fori_loop(0, NSTG, stage, 0)
    plsc.subcore_barrier()
    pltpu.sync_copy(acc.at[pl.ds(sid * RPT, RPT)],
                    out_hbm.at[cid, pl.ds(sid * RPT, RPT)])


@functools.partial(
    pl.kernel,
    out_type=(jax.ShapeDtypeStruct((E,), jnp.float32),
              jax.ShapeDtypeStruct((NW * N,), jnp.float32)),
    mesh=_mesh,
    compiler_params=pltpu.CompilerParams(needs_layout_passes=False),
    scratch_types=[
        pltpu.VMEM((EPW,), jnp.int32),
        pltpu.VMEM((EPW,), jnp.int32),
        pltpu.VMEM((EPW,), jnp.float32),
        pltpu.VMEM((N,), jnp.float32),
        pltpu.VMEM((N,), jnp.float32),
        pltpu.VMEM((N,), jnp.float32),
    ],
)
def _z_kernel(es_hbm, ed_hbm, src_hbm, dst_hbm, z_hbm, den_hbm,
              sidx, didx, zbuf, esv, edv, denv):
    cid = lax.axis_index("c")
    sid = lax.axis_index("s")
    wid = cid * NS + sid
    base = wid * EPW
    pltpu.sync_copy(src_hbm.at[pl.ds(base, EPW)], sidx)
    pltpu.sync_copy(dst_hbm.at[pl.ds(base, EPW)], didx)
    pltpu.sync_copy(es_hbm, esv)
    pltpu.sync_copy(ed_hbm, edv)

    zeros = jnp.zeros((LANE,), jnp.float32)

    def zden(i, carry):
        denv[pl.ds(i * LANE, LANE)] = zeros
        return carry

    lax.fori_loop(0, N // LANE, zden, 0)

    neg = jnp.full((LANE,), -3.0e38, jnp.float32)

    def mx(i, carry):
        a, b = carry
        return (jnp.maximum(a, esv[pl.ds(i * LANE, LANE)]),
                jnp.maximum(b, edv[pl.ds(i * LANE, LANE)]))

    am, bm = lax.fori_loop(0, N // LANE, mx, (neg, neg))
    s_max = jnp.max(am) + jnp.max(bm)
    g_bound = jnp.maximum(s_max, ALPHA * s_max)

    def body(k, carry):
        sl = pl.ds(k * LANE, LANE)
        si = sidx[sl]
        di = didx[sl]
        s = plsc.load_gather(esv, [si]) + plsc.load_gather(edv, [di])
        e = jnp.maximum(s, ALPHA * s)
        z = jnp.exp(e - g_bound)
        zbuf[sl] = z
        plsc.addupdate_scatter(denv, [di], z)
        return carry

    lax.fori_loop(0, EPW // LANE, body, 0)

    pltpu.sync_copy(zbuf, z_hbm.at[pl.ds(base, EPW)])
    pltpu.sync_copy(denv, den_hbm.at[pl.ds(wid * N, N)])


@functools.partial(
    pl.kernel,
    out_type=jax.ShapeDtypeStruct((NC, NPAD, H), jnp.float32),
    mesh=_mesh,
    compiler_params=pltpu.CompilerParams(needs_layout_passes=False),
    scratch_types=[
        pltpu.VMEM_SHARED((NPAD, H), jnp.float32),
        pltpu.VMEM((SST, 128), jnp.int32),
        pltpu.VMEM((SST, 128), jnp.int32),
        pltpu.VMEM((SST, 128), jnp.float32),
        pltpu.VMEM((2, 128, H), jnp.float32),
        pltpu.SemaphoreType.DMA((2,)),
    ],
)
def _wagg_kernel(wh_hbm, src_hbm, dst_hbm, z_hbm, zer_hbm, out_hbm,
                 acc, srows, drows, zrows, rbuf, sem):
    cid = lax.axis_index("c")
    sid = lax.axis_index("s")
    wid = cid * NS + sid
    n_w = jnp.clip(TCH - wid * CPW, 0, CPW)

    pltpu.sync_copy(zer_hbm, acc.at[pl.ds(sid * RPT, RPT)])
    plsc.subcore_barrier()

    def stage(t, carry):
        row0 = wid * CPW + t * SST
        pltpu.sync_copy(src_hbm.at[pl.ds(row0, SST)], srows)
        pltpu.sync_copy(dst_hbm.at[pl.ds(row0, SST)], drows)
        pltpu.sync_copy(z_hbm.at[pl.ds(row0, SST)], zrows)
        n_t = jnp.clip(n_w - t * SST, 0, SST)

        @pl.when(n_t > 0)
        def _():
            pltpu.async_copy(wh_hbm.at[srows.at[0]], rbuf.at[0], sem.at[0])

        def body(j, c2):
            p = lax.rem(j, 2)
            q = lax.rem(j + 1, 2)

            @pl.when(j + 1 < n_t)
            def _():
                pltpu.async_copy(wh_hbm.at[srows.at[j + 1]], rbuf.at[q],
                                 sem.at[q])

            pltpu.make_async_copy(wh_hbm.at[srows.at[j]], rbuf.at[p],
                                  sem.at[p]).wait()
            jv = jnp.full((LANE,), j, jnp.int32)

            def scale(r4, c3):
                for rr in range(4):
                    r = r4 * 4 + rr
                    zr = plsc.load_gather(
                        zrows, [jv, jnp.full((LANE,), r, jnp.int32)])
                    for cc in range(H // LANE):
                        cs = pl.ds(cc * LANE, LANE)
                        rbuf[p, r, cs] = rbuf[p, r, cs] * zr
                return c3

            lax.fori_loop(0, 32, scale, 0)
            pltpu.sync_copy(rbuf.at[p], acc.at[drows.at[j]], add=True)
            return c2

        lax.fori_loop(0, n_t, body, 0)
        return carry

    lax.fori_loop(0, NSTG, stage, 0)
    plsc.subcore_barrier()
    pltpu.sync_copy(acc.at[pl.ds(sid * RPT, RPT)],
                    out_hbm.at[cid, pl.ds(sid * RPT, RPT)])


def _gcn1_tc(x_ref, degp_ref, w1_ref, y_ref):
    deg = jnp.sum(degp_ref[...], axis=0)
    scale = lax.rsqrt(jnp.maximum(deg, 1.0))
    xs = x_ref[...] * scale[:, None]
    y_ref[...] = jnp.dot(xs, w1_ref[...], preferred_element_type=jnp.float32)


def _gat_tc(aggp_ref, degp_ref, w2_ref, as_ref, ad_ref,
            wh_ref, es_ref, ed_ref):
    agg = aggp_ref[0] + aggp_ref[1]
    deg = jnp.sum(degp_ref[...], axis=0)
    h = jnp.maximum(agg * lax.rsqrt(jnp.maximum(deg, 1.0))[:, None], 0.0)
    wh = jnp.dot(h, w2_ref[...], preferred_element_type=jnp.float32)
    wh_ref[...] = wh
    es_ref[...] = jnp.broadcast_to(jnp.dot(wh, as_ref[...])[None, None, :],
                                   (1, 8, B))
    ed_ref[...] = jnp.broadcast_to(jnp.dot(wh, ad_ref[...])[None, None, :],
                                   (1, 8, B))


def _head_tc(h2p_ref, denp_ref, wp_ref, bp_ref, out_ref):
    h2 = h2p_ref[0] + h2p_ref[1]
    den = jnp.sum(denp_ref[...], axis=0)
    h2 = h2 * (1.0 / (den + 1e-16))[:, None]
    h2 = jnp.where(h2 > 0.0, h2, jnp.exp(h2) - 1.0)
    out_ref[...] = (jnp.dot(h2, wp_ref[...], preferred_element_type=jnp.float32)
                    + bp_ref[...][None, :])


def kernel(x, edge_index, W1, W2, a_src, a_dst, Wp, bp):
    src = edge_index[0]
    dst = edge_index[1]
    src2d = jnp.pad(src.reshape(TCH, 128), ((0, PADROWS - TCH), (0, 0)))
    dst2d = jnp.pad(dst.reshape(TCH, 128), ((0, PADROWS - TCH), (0, 0)))
    zer = jnp.zeros((RPT, H), jnp.float32)

    dof, dif = _deg_kernel(src, dst)
    dop = dof.reshape(NW, N)
    dip = dif.reshape(NW, N)

    y = pl.pallas_call(
        _gcn1_tc,
        grid=(G,),
        in_specs=[
            pl.BlockSpec((B, D), lambda g: (g, 0)),
            pl.BlockSpec((NW, B), lambda g: (0, g)),
            pl.BlockSpec((D, H), lambda g: (0, 0)),
        ],
        out_specs=pl.BlockSpec((B, H), lambda g: (g, 0)),
        out_shape=jax.ShapeDtypeStruct((N, H), jnp.float32),
    )(x, dop, W1)

    aggp = _agg_kernel(y, src2d, dst2d, zer)

    wh, es2, ed2 = pl.pallas_call(
        _gat_tc,
        grid=(G,),
        in_specs=[
            pl.BlockSpec((NC, B, H), lambda g: (0, g, 0)),
            pl.BlockSpec((NW, B), lambda g: (0, g)),
            pl.BlockSpec((H, H), lambda g: (0, 0)),
            pl.BlockSpec((H,), lambda g: (0,)),
            pl.BlockSpec((H,), lambda g: (0,)),
        ],
        out_specs=[
            pl.BlockSpec((B, H), lambda g: (g, 0)),
            pl.BlockSpec((1, 8, B), lambda g: (g, 0, 0)),
            pl.BlockSpec((1, 8, B), lambda g: (g, 0, 0)),
        ],
        out_shape=[
            jax.ShapeDtypeStruct((N, H), jnp.float32),
            jax.ShapeDtypeStruct((G, 8, B), jnp.float32),
            jax.ShapeDtypeStruct((G, 8, B), jnp.float32),
        ],
    )(aggp, dip, W2, a_src, a_dst)

    es = es2[:, 0, :].reshape(G * B)[:N]
    ed = ed2[:, 0, :].reshape(G * B)[:N]

    zf, denf = _z_kernel(es, ed, src, dst)
    denp = denf.reshape(NW, N)
    z2d = jnp.pad(zf.reshape(TCH, 128), ((0, PADROWS - TCH), (0, 0)))

    h2p = _wagg_kernel(wh, src2d, dst2d, z2d, zer)

    logits = pl.pallas_call(
        _head_tc,
        grid=(G,),
        in_specs=[
            pl.BlockSpec((NC, B, H), lambda g: (0, g, 0)),
            pl.BlockSpec((NW, B), lambda g: (0, g)),
            pl.BlockSpec((H, C), lambda g: (0, 0)),
            pl.BlockSpec((C,), lambda g: (0,)),
        ],
        out_specs=pl.BlockSpec((B, C), lambda g: (g, 0)),
        out_shape=jax.ShapeDtypeStruct((N, C), jnp.float32),
    )(h2p, denp, Wp, bp)

    return logits


# speedup vs baseline: 1.1666x; 1.0063x over previous
import functools

import jax
import jax.numpy as jnp
from jax import lax
from jax.experimental import pallas as pl
from jax.experimental.pallas import tpu as pltpu
from jax.experimental.pallas import tpu_sc as plsc

N = 10000
E = 320000
D = 128
H = 128
C = 10
ALPHA = 0.1

NC = 2
NS = 16
NW = NC * NS
LANE = 16

EPW = E // NW
TCH = E // 128
CPW = 80
SST = 16
NSTG = CPW // SST
PADROWS = CPW * NW
NPAD = 10240
RPT = NPAD // NS

B = 1024
G = -(-N // B)

_mesh = plsc.VectorSubcoreMesh(
    core_axis_name="c", subcore_axis_name="s", num_cores=NC, num_subcores=NS)


@functools.partial(
    pl.kernel,
    out_type=(jax.ShapeDtypeStruct((NW * N,), jnp.float32),
              jax.ShapeDtypeStruct((NW * N,), jnp.float32)),
    mesh=_mesh,
    compiler_params=pltpu.CompilerParams(needs_layout_passes=False),
    scratch_types=[
        pltpu.VMEM((EPW,), jnp.int32),
        pltpu.VMEM((EPW,), jnp.int32),
        pltpu.VMEM((N,), jnp.float32),
        pltpu.VMEM((N,), jnp.float32),
    ],
)
def _deg_kernel(src_hbm, dst_hbm, do_hbm, di_hbm, sidx, didx, dout, din):
    cid = lax.axis_index("c")
    sid = lax.axis_index("s")
    wid = cid * NS + sid
    base = wid * EPW
    pltpu.sync_copy(src_hbm.at[pl.ds(base, EPW)], sidx)
    pltpu.sync_copy(dst_hbm.at[pl.ds(base, EPW)], didx)

    zeros = jnp.zeros((LANE,), jnp.float32)

    def zero_body(i, carry):
        dout[pl.ds(i * LANE, LANE)] = zeros
        din[pl.ds(i * LANE, LANE)] = zeros
        return carry

    lax.fori_loop(0, N // LANE, zero_body, 0)

    ones = jnp.ones((LANE,), jnp.float32)

    def body(k, carry):
        si = sidx[pl.ds(k * LANE, LANE)]
        plsc.addupdate_scatter(dout, [si], ones)
        di = didx[pl.ds(k * LANE, LANE)]
        plsc.addupdate_scatter(din, [di], ones)
        return carry

    lax.fori_loop(0, EPW // LANE, body, 0)

    pltpu.sync_copy(dout, do_hbm.at[pl.ds(wid * N, N)])
    pltpu.sync_copy(din, di_hbm.at[pl.ds(wid * N, N)])


@functools.partial(
    pl.kernel,
    out_type=jax.ShapeDtypeStruct((NC, NPAD, H), jnp.float32),
    mesh=_mesh,
    compiler_params=pltpu.CompilerParams(needs_layout_passes=False),
    scratch_types=[
        pltpu.VMEM_SHARED((NPAD, H), jnp.float32),
        pltpu.VMEM((SST, 128), jnp.int32),
        pltpu.VMEM((SST, 128), jnp.int32),
        pltpu.VMEM((2, 128, H), jnp.float32),
        pltpu.SemaphoreType.DMA((2,)),
    ],
)
def _agg_kernel(y_hbm, src_hbm, dst_hbm, zer_hbm, out_hbm,
                acc, srows, drows, rbuf, sem):
    cid = lax.axis_index("c")
    sid = lax.axis_index("s")
    wid = cid * NS + sid
    n_w = jnp.clip(TCH - wid * CPW, 0, CPW)

    pltpu.sync_copy(zer_hbm, acc.at[pl.ds(sid * RPT, RPT)])
    plsc.subcore_barrier()

    def stage(t, carry):
        row0 = wid * CPW + t * SST
        pltpu.sync_copy(src_hbm.at[pl.ds(row0, SST)], srows)
        pltpu.sync_copy(dst_hbm.at[pl.ds(row0, SST)], drows)
        n_t = jnp.clip(n_w - t * SST, 0, SST)

        @pl.when(n_t > 0)
        def _():
            pltpu.async_copy(y_hbm.at[srows.at[0]], rbuf.at[0], sem.at[0])

        def body(j, c2):
            p = lax.rem(j, 2)
            q = lax.rem(j + 1, 2)

            @pl.when(j + 1 < n_t)
            def _():
                pltpu.async_copy(y_hbm.at[srows.at[j + 1]], rbuf.at[q],
                                 sem.at[q])

            pltpu.make_async_copy(y_hbm.at[srows.at[j]], rbuf.at[p],
                                  sem.at[p]).wait()
            pltpu.sync_copy(rbuf.at[p], acc.at[drows.at[j]], add=True)
            return c2

        lax.fori_loop(0, n_t, body, 0)
        return carry

    lax.fori_loop(0, NSTG, stage, 0)
    plsc.subcore_barrier()
    pltpu.sync_copy(acc.at[pl.ds(sid * RPT, RPT)],
                    out_hbm.at[cid, pl.ds(sid * RPT, RPT)])


@functools.partial(
    pl.kernel,
    out_type=(jax.ShapeDtypeStruct((PADROWS * 128,), jnp.float32),
              jax.ShapeDtypeStruct((NW * N,), jnp.float32)),
    mesh=_mesh,
    compiler_params=pltpu.CompilerParams(needs_layout_passes=False),
    scratch_types=[
        pltpu.VMEM((EPW,), jnp.int32),
        pltpu.VMEM((EPW,), jnp.int32),
        pltpu.VMEM((EPW,), jnp.float32),
        pltpu.VMEM((N,), jnp.float32),
        pltpu.VMEM((N,), jnp.float32),
        pltpu.VMEM((N,), jnp.float32),
    ],
)
def _z_kernel(es_hbm, ed_hbm, src_hbm, dst_hbm, z_hbm, den_hbm,
              sidx, didx, zbuf, esv, edv, denv):
    cid = lax.axis_index("c")
    sid = lax.axis_index("s")
    wid = cid * NS + sid
    base = wid * EPW
    pltpu.sync_copy(src_hbm.at[pl.ds(base, EPW)], sidx)
    pltpu.sync_copy(dst_hbm.at[pl.ds(base, EPW)], didx)
    pltpu.sync_copy(es_hbm, esv)
    pltpu.sync_copy(ed_hbm, edv)

    zeros = jnp.zeros((LANE,), jnp.float32)
    neg = jnp.full((LANE,), -3.0e38, jnp.float32)

    def mx(i, carry):
        a, b = carry
        denv[pl.ds(i * LANE, LANE)] = zeros
        return (jnp.maximum(a, esv[pl.ds(i * LANE, LANE)]),
                jnp.maximum(b, edv[pl.ds(i * LANE, LANE)]))

    am, bm = lax.fori_loop(0, N // LANE, mx, (neg, neg))
    s_max = jnp.max(am) + jnp.max(bm)
    g_bound = jnp.maximum(s_max, ALPHA * s_max)

    def body(k, carry):
        sl = pl.ds(k * LANE, LANE)
        si = sidx[sl]
        di = didx[sl]
        s = plsc.load_gather(esv, [si]) + plsc.load_gather(edv, [di])
        e = jnp.maximum(s, ALPHA * s)
        z = jnp.exp(e - g_bound)
        zbuf[sl] = z
        plsc.addupdate_scatter(denv, [di], z)
        return carry

    lax.fori_loop(0, EPW // LANE, body, 0)

    pltpu.sync_copy(zbuf, z_hbm.at[pl.ds(base, EPW)])
    pltpu.sync_copy(denv, den_hbm.at[pl.ds(wid * N, N)])


@functools.partial(
    pl.kernel,
    out_type=jax.ShapeDtypeStruct((NC, NPAD, H), jnp.float32),
    mesh=_mesh,
    compiler_params=pltpu.CompilerParams(needs_layout_passes=False),
    scratch_types=[
        pltpu.VMEM_SHARED((NPAD, H), jnp.float32),
        pltpu.VMEM((SST, 128), jnp.int32),
        pltpu.VMEM((SST, 128), jnp.int32),
        pltpu.VMEM((SST, 128), jnp.float32),
        pltpu.VMEM((2, 128, H), jnp.float32),
        pltpu.SemaphoreType.DMA((2,)),
    ],
)
def _wagg_kernel(wh_hbm, src_hbm, dst_hbm, z_hbm, zer_hbm, out_hbm,
                 acc, srows, drows, zrows, rbuf, sem):
    cid = lax.axis_index("c")
    sid = lax.axis_index("s")
    wid = cid * NS + sid
    n_w = jnp.clip(TCH - wid * CPW, 0, CPW)

    pltpu.sync_copy(zer_hbm, acc.at[pl.ds(sid * RPT, RPT)])
    plsc.subcore_barrier()

    def stage(t, carry):
        row0 = wid * CPW + t * SST
        pltpu.sync_copy(src_hbm.at[pl.ds(row0, SST)], srows)
        pltpu.sync_copy(dst_hbm.at[pl.ds(row0, SST)], drows)
        pltpu.sync_copy(z_hbm.at[pl.ds(row0, SST)], zrows)
        n_t = jnp.clip(n_w - t * SST, 0, SST)

        @pl.when(n_t > 0)
        def _():
            pltpu.async_copy(wh_hbm.at[srows.at[0]], rbuf.at[0], sem.at[0])

        def body(j, c2):
            p = lax.rem(j, 2)
            q = lax.rem(j + 1, 2)

            @pl.when(j + 1 < n_t)
            def _():
                pltpu.async_copy(wh_hbm.at[srows.at[j + 1]], rbuf.at[q],
                                 sem.at[q])

            pltpu.make_async_copy(wh_hbm.at[srows.at[j]], rbuf.at[p],
                                  sem.at[p]).wait()
            jv = jnp.full((LANE,), j, jnp.int32)

            def scale(r8, c3):
                for rr in range(8):
                    r = r8 * 8 + rr
                    zr = plsc.load_gather(
                        zrows, [jv, jnp.full((LANE,), r, jnp.int32)])
                    for cc in range(H // LANE):
                        cs = pl.ds(cc * LANE, LANE)
                        rbuf[p, r, cs] = rbuf[p, r, cs] * zr
                return c3

            lax.fori_loop(0, 16, scale, 0)
            pltpu.sync_copy(rbuf.at[p], acc.at[drows.at[j]], add=True)
            return c2

        lax.fori_loop(0, n_t, body, 0)
        return carry

    lax.fori_loop(0, NSTG, stage, 0)
    plsc.subcore_barrier()
    pltpu.sync_copy(acc.at[pl.ds(sid * RPT, RPT)],
                    out_hbm.at[cid, pl.ds(sid * RPT, RPT)])


def _gcn1_tc(x_ref, degp_ref, w1_ref, y_ref):
    deg = jnp.sum(degp_ref[...], axis=0)
    scale = lax.rsqrt(jnp.maximum(deg, 1.0))
    xs = x_ref[...] * scale[:, None]
    y_ref[...] = jnp.dot(xs, w1_ref[...], preferred_element_type=jnp.float32)


def _gat_tc(aggp_ref, degp_ref, w2_ref, as_ref, ad_ref,
            wh_ref, es_ref, ed_ref):
    agg = aggp_ref[0] + aggp_ref[1]
    deg = jnp.sum(degp_ref[...], axis=0)
    h = jnp.maximum(agg * lax.rsqrt(jnp.maximum(deg, 1.0))[:, None], 0.0)
    wh = jnp.dot(h, w2_ref[...], preferred_element_type=jnp.float32)
    wh_ref[...] = wh
    es_ref[...] = jnp.broadcast_to(jnp.dot(wh, as_ref[...])[None, None, :],
                                   (1, 8, B))
    ed_ref[...] = jnp.broadcast_to(jnp.dot(wh, ad_ref[...])[None, None, :],
                                   (1, 8, B))


def _head_tc(h2p_ref, denp_ref, wp_ref, bp_ref, out_ref):
    h2 = h2p_ref[0] + h2p_ref[1]
    den = jnp.sum(denp_ref[...], axis=0)
    h2 = h2 * (1.0 / (den + 1e-16))[:, None]
    h2 = jnp.where(h2 > 0.0, h2, jnp.exp(h2) - 1.0)
    out_ref[...] = (jnp.dot(h2, wp_ref[...], preferred_element_type=jnp.float32)
                    + bp_ref[...][None, :])


def kernel(x, edge_index, W1, W2, a_src, a_dst, Wp, bp):
    src = edge_index[0]
    dst = edge_index[1]
    src2d = jnp.pad(src.reshape(TCH, 128), ((0, PADROWS - TCH), (0, 0)))
    dst2d = jnp.pad(dst.reshape(TCH, 128), ((0, PADROWS - TCH), (0, 0)))
    zer = jnp.zeros((RPT, H), jnp.float32)

    dof, dif = _deg_kernel(src, dst)
    dop = dof.reshape(NW, N)
    dip = dif.reshape(NW, N)

    y = pl.pallas_call(
        _gcn1_tc,
        grid=(G,),
        in_specs=[
            pl.BlockSpec((B, D), lambda g: (g, 0)),
            pl.BlockSpec((NW, B), lambda g: (0, g)),
            pl.BlockSpec((D, H), lambda g: (0, 0)),
        ],
        out_specs=pl.BlockSpec((B, H), lambda g: (g, 0)),
        out_shape=jax.ShapeDtypeStruct((N, H), jnp.float32),
    )(x, dop, W1)

    aggp = _agg_kernel(y, src2d, dst2d, zer)

    wh, es2, ed2 = pl.pallas_call(
        _gat_tc,
        grid=(G,),
        in_specs=[
            pl.BlockSpec((NC, B, H), lambda g: (0, g, 0)),
            pl.BlockSpec((NW, B), lambda g: (0, g)),
            pl.BlockSpec((H, H), lambda g: (0, 0)),
            pl.BlockSpec((H,), lambda g: (0,)),
            pl.BlockSpec((H,), lambda g: (0,)),
        ],
        out_specs=[
            pl.BlockSpec((B, H), lambda g: (g, 0)),
            pl.BlockSpec((1, 8, B), lambda g: (g, 0, 0)),
            pl.BlockSpec((1, 8, B), lambda g: (g, 0, 0)),
        ],
        out_shape=[
            jax.ShapeDtypeStruct((N, H), jnp.float32),
            jax.ShapeDtypeStruct((G, 8, B), jnp.float32),
            jax.ShapeDtypeStruct((G, 8, B), jnp.float32),
        ],
    )(aggp, dip, W2, a_src, a_dst)

    es = es2[:, 0, :].reshape(G * B)[:N]
    ed = ed2[:, 0, :].reshape(G * B)[:N]

    zf, denf = _z_kernel(es, ed, src, dst)
    denp = denf.reshape(NW, N)
    z2d = zf.reshape(PADROWS, 128)

    h2p = _wagg_kernel(wh, src2d, dst2d, z2d, zer)

    logits = pl.pallas_call(
        _head_tc,
        grid=(G,),
        in_specs=[
            pl.BlockSpec((NC, B, H), lambda g: (0, g, 0)),
            pl.BlockSpec((NW, B), lambda g: (0, g)),
            pl.BlockSpec((H, C), lambda g: (0, 0)),
            pl.BlockSpec((C,), lambda g: (0,)),
        ],
        out_specs=pl.BlockSpec((B, C), lambda g: (g, 0)),
        out_shape=jax.ShapeDtypeStruct((N, C), jnp.float32),
    )(h2p, denp, Wp, bp)

    return logits
